# baseline (speedup 1.0000x reference)
import functools

import jax
import jax.numpy as jnp
from jax import lax
from jax.experimental import pallas as pl
from jax.experimental.pallas import tpu as pltpu
from jax.experimental.pallas import tpu_sc as plsc

N = 10000
E = 320000
D_IN = 128
D_H = 256
HALF = 128
EPS = 1e-5

NC = 2
NS = 16
CHUNK = 128
BLK = 16
NBLK = 10
N_CHUNKS = NBLK * BLK
EPT = N_CHUNKS * CHUNK
E_PAD = NS * EPT
ACC_ROWS = 10240
ZR = ACC_ROWS // NS

DEG_CHUNK = 64
DEG_EPT = E_PAD // (NC * NS)
DEG_NCHUNKS = DEG_EPT // DEG_CHUNK


def _sc_mesh():
  return plsc.VectorSubcoreMesh(
      core_axis_name="c", subcore_axis_name="s", num_cores=NC, num_subcores=NS)


@functools.partial(
    pl.kernel,
    out_type=jax.ShapeDtypeStruct((NC, ACC_ROWS, HALF), jnp.float32),
    mesh=_sc_mesh(),
    scratch_types=[
        pltpu.VMEM((DEG_CHUNK,), jnp.int32),
        pltpu.VMEM((DEG_CHUNK, HALF), jnp.float32),
        pltpu.VMEM_SHARED((ACC_ROWS, HALF), jnp.float32),
    ],
)
def _sc_degree(dst_hbm, zero_hbm, out_hbm, didx, ones, acc):
  c = lax.axis_index("c")
  s = lax.axis_index("s")

  pltpu.sync_copy(zero_hbm.at[pl.ds(0, ZR)], acc.at[pl.ds(s * ZR, ZR)])

  @pl.loop(0, DEG_CHUNK)
  def _fill(r):
    for j in range(HALF // 16):
      ones[r, pl.ds(j * 16, 16)] = jnp.full((16,), 1.0, dtype=jnp.float32)

  plsc.subcore_barrier()

  base = (c * NS + s) * DEG_EPT

  @pl.loop(0, DEG_NCHUNKS)
  def _body(g):
    off = base + g * DEG_CHUNK
    pltpu.sync_copy(dst_hbm.at[pl.ds(off, DEG_CHUNK)], didx)
    pltpu.sync_copy(ones, acc.at[didx], add=True)

  plsc.subcore_barrier()
  pltpu.sync_copy(acc.at[pl.ds(s * ZR, ZR)], out_hbm.at[c, pl.ds(s * ZR, ZR)])


@functools.partial(
    pl.kernel,
    out_type=jax.ShapeDtypeStruct((NC * ACC_ROWS, HALF), jnp.float32),
    mesh=_sc_mesh(),
    scratch_types=[
        pltpu.VMEM((2, BLK, CHUNK), jnp.int32),
        pltpu.VMEM((2, BLK, CHUNK), jnp.int32),
        pltpu.VMEM((2, CHUNK, HALF), jnp.float32),
        pltpu.VMEM_SHARED((ACC_ROWS, HALF), jnp.float32),
        pltpu.SemaphoreType.DMA,
        pltpu.SemaphoreType.DMA,
        pltpu.SemaphoreType.DMA,
        pltpu.SemaphoreType.DMA,
        pltpu.SemaphoreType.DMA,
        pltpu.SemaphoreType.DMA,
    ],
)
def _sc_aggregate(h2_hbm, src_hbm, dst_hbm, zero_hbm, out_hbm,
                  sidx, didx, rows, acc, gsem0, gsem1, ssem0, ssem1,
                  isem0, isem1):
  c = lax.axis_index("c")
  s = lax.axis_index("s")
  gsems = (gsem0, gsem1)
  ssems = (ssem0, ssem1)
  isems = (isem0, isem1)

  pltpu.sync_copy(zero_hbm.at[pl.ds(0, ZR)], acc.at[pl.ds(s * ZR, ZR)])
  pltpu.sync_copy(src_hbm.at[c, s, 0], sidx.at[0])
  pltpu.sync_copy(dst_hbm.at[s, 0], didx.at[0])
  pltpu.async_copy(src_hbm.at[c, s, 1], sidx.at[1], isem1)
  pltpu.async_copy(dst_hbm.at[s, 1], didx.at[1], isem1)
  plsc.subcore_barrier()
  pltpu.async_copy(h2_hbm.at[sidx.at[0, 0]], rows.at[0], gsem0)

  @pl.loop(0, NBLK, step=2)
  def _outer(k0):
    for kb in range(2):
      blk = k0 + kb
      for j in range(BLK):
        b = j % 2
        pltpu.make_async_copy(h2_hbm.at[sidx.at[kb, j]], rows.at[b],
                              gsems[b]).wait()
        pltpu.async_copy(rows.at[b], acc.at[didx.at[kb, j]], ssems[b],
                         add=True)

        if j == 0:
          @pl.when(blk > 0)
          def _wait_prev_tail():
            pltpu.make_async_copy(rows.at[1], acc.at[didx.at[1 - kb, 0]],
                                  ssems[1]).wait()

          @pl.when((blk >= 1) & (blk + 1 < NBLK))
          def _refill():
            pltpu.async_copy(src_hbm.at[c, s, blk + 1], sidx.at[1 - kb],
                             isems[1 - kb])
            pltpu.async_copy(dst_hbm.at[s, blk + 1], didx.at[1 - kb],
                             isems[1 - kb])
        else:
          pltpu.make_async_copy(rows.at[1 - b], acc.at[didx.at[kb, j - 1]],
                                ssems[1 - b]).wait()

        if j < BLK - 1:
          pltpu.async_copy(h2_hbm.at[sidx.at[kb, j + 1]], rows.at[1 - b],
                           gsems[1 - b])
        else:
          @pl.when(blk + 1 < NBLK)
          def _prefetch_next_block():
            pltpu.make_async_copy(src_hbm.at[c, s, blk + 1],
                                  sidx.at[1 - kb], isems[1 - kb]).wait()
            pltpu.make_async_copy(dst_hbm.at[s, blk + 1],
                                  didx.at[1 - kb], isems[1 - kb]).wait()
            pltpu.async_copy(h2_hbm.at[sidx.at[1 - kb, 0]], rows.at[1 - b],
                             gsems[1 - b])

  pltpu.make_async_copy(rows.at[1], acc.at[didx.at[1, BLK - 1]],
                        ssems[1]).wait()
  plsc.subcore_barrier()

  pltpu.sync_copy(acc.at[pl.ds(s * ZR, ZR)],
                  out_hbm.at[pl.ds(c * ACC_ROWS + s * ZR, ZR)])


def _tc_matmul0(x, w0, deg2):
  def body(x_ref, w_ref, deg_ref, h2_ref):
    deg = deg_ref[0, :N, 0] + deg_ref[1, :N, 0] + 1.0
    dinv = lax.rsqrt(deg)
    h = jnp.dot(x_ref[...], w_ref[...], preferred_element_type=jnp.float32)
    hp = h * dinv[:, None]
    h2_ref[0, :, :] = hp[:, :HALF]
    h2_ref[1, :, :] = hp[:, HALF:]

  return pl.pallas_call(
      body,
      out_shape=jax.ShapeDtypeStruct((NC, N, HALF), jnp.float32),
  )(x, w0, deg2)


def _tc_layer(agg2, h2prev, deg2, b, g, be, w_next):
  final = w_next is None

  def body(agg_ref, hprev_ref, deg_ref, b_ref, g_ref, be_ref, *rest):
    if final:
      (out_ref,) = rest
    else:
      w_ref, out_ref = rest
    deg = deg_ref[0, :N, 0] + deg_ref[1, :N, 0] + 1.0
    dinv = lax.rsqrt(deg)

    halves = []
    for h_i in range(NC):
      z = (agg_ref[h_i, :N, :] + hprev_ref[h_i]) * dinv[:, None]
      z = z + b_ref[pl.ds(h_i * HALF, HALF)][None, :]
      mu = jnp.mean(z, axis=0)
      zc = z - mu[None, :]
      var = jnp.mean(zc * zc, axis=0)
      zn = (g_ref[pl.ds(h_i * HALF, HALF)][None, :] * zc
            * lax.rsqrt(var + EPS)[None, :]
            + be_ref[pl.ds(h_i * HALF, HALF)][None, :])
      halves.append(jnp.maximum(zn, 0.0))

    if final:
      out_ref[:, :HALF] = halves[0]
      out_ref[:, HALF:] = halves[1]
    else:
      hn = (jnp.dot(halves[0], w_ref[:HALF, :],
                    preferred_element_type=jnp.float32)
            + jnp.dot(halves[1], w_ref[HALF:, :],
                      preferred_element_type=jnp.float32))
      hp = hn * dinv[:, None]
      out_ref[0, :, :] = hp[:, :HALF]
      out_ref[1, :, :] = hp[:, HALF:]

  if final:
    out_shape = jax.ShapeDtypeStruct((N, D_H), jnp.float32)
    args = (agg2, h2prev, deg2, b, g, be)
  else:
    out_shape = jax.ShapeDtypeStruct((NC, N, HALF), jnp.float32)
    args = (agg2, h2prev, deg2, b, g, be, w_next)

  return pl.pallas_call(body, out_shape=out_shape)(*args)


def kernel(x, edge_index, W0, b0, g0, be0, W1, b1, g1, be1, W2, b2, g2, be2):
  src = edge_index[0]
  dst = edge_index[1]
  pad = E_PAD - E
  src_p = jnp.concatenate([src, jnp.zeros((pad,), jnp.int32)])
  pad_dst = N + jnp.arange(pad, dtype=jnp.int32) % (ACC_ROWS - N)
  dst_p = jnp.concatenate([dst, pad_dst])
  src2 = jnp.stack([src_p, src_p + N]).reshape(NC, NS, NBLK, BLK, CHUNK)
  dst_agg = dst_p.reshape(NS, NBLK, BLK, CHUNK)

  zeroH = jnp.zeros((ZR, HALF), jnp.float32)

  deg2 = _sc_degree(dst_p, zeroH)

  h2 = _tc_matmul0(x, W0, deg2)
  params = [(b0, g0, be0, W1), (b1, g1, be1, W2), (b2, g2, be2, None)]
  for b, g, be, w_next in params:
    aggflat = _sc_aggregate(h2.reshape(NC * N, HALF), src2, dst_agg, zeroH)
    h2 = _tc_layer(aggflat.reshape(NC, ACC_ROWS, HALF), h2, deg2, b, g, be,
                   w_next)
  return h2


# speedup vs baseline: 1.0603x; 1.0603x over previous
import functools

import jax
import jax.numpy as jnp
from jax import lax
from jax.experimental import pallas as pl
from jax.experimental.pallas import tpu as pltpu
from jax.experimental.pallas import tpu_sc as plsc

N = 10000
E = 320000
D_IN = 128
D_H = 256
HALF = 128
EPS = 1e-5

NC = 2
NS = 16
CHUNK = 128
BLK = 16
NBLK = 10
N_CHUNKS = NBLK * BLK
EPT = N_CHUNKS * CHUNK
E_PAD = NS * EPT
ACC_ROWS = 10240
ZR = ACC_ROWS // NS

DEG_CHUNK = 128
DEG_EPT = E_PAD // (NC * NS)
DEG_NCHUNKS = DEG_EPT // DEG_CHUNK


def _sc_mesh():
  return plsc.VectorSubcoreMesh(
      core_axis_name="c", subcore_axis_name="s", num_cores=NC, num_subcores=NS)


@functools.partial(
    pl.kernel,
    out_type=jax.ShapeDtypeStruct((NC, ACC_ROWS, HALF), jnp.float32),
    mesh=_sc_mesh(),
    scratch_types=[
        pltpu.VMEM((DEG_NCHUNKS, DEG_CHUNK), jnp.int32),
        pltpu.VMEM((DEG_CHUNK, HALF), jnp.float32),
        pltpu.VMEM_SHARED((ACC_ROWS, HALF), jnp.float32),
        pltpu.SemaphoreType.DMA,
    ],
)
def _sc_degree(dst_hbm, zero_hbm, out_hbm, didx, ones, acc, ssem):
  c = lax.axis_index("c")
  s = lax.axis_index("s")

  pltpu.sync_copy(zero_hbm.at[pl.ds(0, ZR)], acc.at[pl.ds(s * ZR, ZR)])
  pltpu.sync_copy(dst_hbm.at[c, s], didx)

  @pl.loop(0, DEG_CHUNK)
  def _fill(r):
    for j in range(HALF // 16):
      ones[r, pl.ds(j * 16, 16)] = jnp.full((16,), 1.0, dtype=jnp.float32)

  plsc.subcore_barrier()

  for g in range(DEG_NCHUNKS):
    pltpu.async_copy(ones, acc.at[didx.at[g]], ssem, add=True)
  for g in range(DEG_NCHUNKS):
    pltpu.make_async_copy(ones, acc.at[didx.at[g]], ssem).wait()

  plsc.subcore_barrier()
  pltpu.sync_copy(acc.at[pl.ds(s * ZR, ZR)], out_hbm.at[c, pl.ds(s * ZR, ZR)])


@functools.partial(
    pl.kernel,
    out_type=jax.ShapeDtypeStruct((NC * ACC_ROWS, HALF), jnp.float32),
    mesh=_sc_mesh(),
    scratch_types=[
        pltpu.VMEM((2, BLK, CHUNK), jnp.int32),
        pltpu.VMEM((2, BLK, CHUNK), jnp.int32),
        pltpu.VMEM((2, CHUNK, HALF), jnp.float32),
        pltpu.VMEM_SHARED((ACC_ROWS, HALF), jnp.float32),
        pltpu.SemaphoreType.DMA,
        pltpu.SemaphoreType.DMA,
        pltpu.SemaphoreType.DMA,
        pltpu.SemaphoreType.DMA,
        pltpu.SemaphoreType.DMA,
        pltpu.SemaphoreType.DMA,
    ],
)
def _sc_aggregate(h2_hbm, src_hbm, dst_hbm, zero_hbm, out_hbm,
                  sidx, didx, rows, acc, gsem0, gsem1, ssem0, ssem1,
                  isem0, isem1):
  c = lax.axis_index("c")
  s = lax.axis_index("s")
  gsems = (gsem0, gsem1)
  ssems = (ssem0, ssem1)
  isems = (isem0, isem1)

  pltpu.sync_copy(zero_hbm.at[pl.ds(0, ZR)], acc.at[pl.ds(s * ZR, ZR)])
  pltpu.sync_copy(src_hbm.at[c, s, 0], sidx.at[0])
  pltpu.sync_copy(dst_hbm.at[s, 0], didx.at[0])
  pltpu.async_copy(src_hbm.at[c, s, 1], sidx.at[1], isem1)
  pltpu.async_copy(dst_hbm.at[s, 1], didx.at[1], isem1)
  plsc.subcore_barrier()
  pltpu.async_copy(h2_hbm.at[sidx.at[0, 0]], rows.at[0], gsem0)

  @pl.loop(0, NBLK, step=2)
  def _outer(k0):
    for kb in range(2):
      blk = k0 + kb
      for j in range(BLK):
        b = j % 2
        pltpu.make_async_copy(h2_hbm.at[sidx.at[kb, j]], rows.at[b],
                              gsems[b]).wait()
        pltpu.async_copy(rows.at[b], acc.at[didx.at[kb, j]], ssems[b],
                         add=True)

        if j == 0:
          @pl.when(blk > 0)
          def _wait_prev_tail():
            pltpu.make_async_copy(rows.at[1], acc.at[didx.at[1 - kb, 0]],
                                  ssems[1]).wait()

          @pl.when((blk >= 1) & (blk + 1 < NBLK))
          def _refill():
            pltpu.async_copy(src_hbm.at[c, s, blk + 1], sidx.at[1 - kb],
                             isems[1 - kb])
            pltpu.async_copy(dst_hbm.at[s, blk + 1], didx.at[1 - kb],
                             isems[1 - kb])
        else:
          pltpu.make_async_copy(rows.at[1 - b], acc.at[didx.at[kb, j - 1]],
                                ssems[1 - b]).wait()

        if j < BLK - 1:
          pltpu.async_copy(h2_hbm.at[sidx.at[kb, j + 1]], rows.at[1 - b],
                           gsems[1 - b])
        else:
          @pl.when(blk + 1 < NBLK)
          def _prefetch_next_block():
            pltpu.make_async_copy(src_hbm.at[c, s, blk + 1],
                                  sidx.at[1 - kb], isems[1 - kb]).wait()
            pltpu.make_async_copy(dst_hbm.at[s, blk + 1],
                                  didx.at[1 - kb], isems[1 - kb]).wait()
            pltpu.async_copy(h2_hbm.at[sidx.at[1 - kb, 0]], rows.at[1 - b],
                             gsems[1 - b])

  pltpu.make_async_copy(rows.at[1], acc.at[didx.at[1, BLK - 1]],
                        ssems[1]).wait()
  plsc.subcore_barrier()

  pltpu.sync_copy(acc.at[pl.ds(s * ZR, ZR)],
                  out_hbm.at[pl.ds(c * ACC_ROWS + s * ZR, ZR)])


def _tc_matmul0(x, w0):
  def body(x_ref, w_ref, h_ref):
    h_ref[...] = jnp.dot(x_ref[...], w_ref[...],
                         preferred_element_type=jnp.float32)

  return pl.pallas_call(
      body,
      out_shape=jax.ShapeDtypeStruct((N, D_H), jnp.float32),
  )(x, w0)


def _tc_scale_split(h0, deg2):
  def body(h_ref, deg_ref, h2_ref, dinv_ref):
    deg = deg_ref[0, :N, 0] + deg_ref[1, :N, 0] + 1.0
    dinv = lax.rsqrt(deg)
    hp = h_ref[...] * dinv[:, None]
    h2_ref[0, :, :] = hp[:, :HALF]
    h2_ref[1, :, :] = hp[:, HALF:]
    dinv_ref[...] = jnp.broadcast_to(dinv[None, :], (8, N))

  return pl.pallas_call(
      body,
      out_shape=(jax.ShapeDtypeStruct((NC, N, HALF), jnp.float32),
                 jax.ShapeDtypeStruct((8, N), jnp.float32)),
  )(h0, deg2)


def _tc_layer(agg2, h2prev, dinv_row, b, g, be, w_next):
  final = w_next is None

  def body(agg_ref, hprev_ref, dinv_ref, b_ref, g_ref, be_ref, *rest):
    if final:
      (out_ref,) = rest
    else:
      w_ref, out_ref = rest
    dinv = dinv_ref[0, :]

    halves = []
    for h_i in range(NC):
      z = (agg_ref[h_i, :N, :] + hprev_ref[h_i]) * dinv[:, None]
      z = z + b_ref[pl.ds(h_i * HALF, HALF)][None, :]
      mu = jnp.mean(z, axis=0)
      zc = z - mu[None, :]
      var = jnp.mean(zc * zc, axis=0)
      zn = (g_ref[pl.ds(h_i * HALF, HALF)][None, :] * zc
            * lax.rsqrt(var + EPS)[None, :]
            + be_ref[pl.ds(h_i * HALF, HALF)][None, :])
      halves.append(jnp.maximum(zn, 0.0))

    if final:
      out_ref[:, :HALF] = halves[0]
      out_ref[:, HALF:] = halves[1]
    else:
      hn = (jnp.dot(halves[0], w_ref[:HALF, :],
                    preferred_element_type=jnp.float32)
            + jnp.dot(halves[1], w_ref[HALF:, :],
                      preferred_element_type=jnp.float32))
      hp = hn * dinv[:, None]
      out_ref[0, :, :] = hp[:, :HALF]
      out_ref[1, :, :] = hp[:, HALF:]

  if final:
    out_shape = jax.ShapeDtypeStruct((N, D_H), jnp.float32)
    args = (agg2, h2prev, dinv_row, b, g, be)
  else:
    out_shape = jax.ShapeDtypeStruct((NC, N, HALF), jnp.float32)
    args = (agg2, h2prev, dinv_row, b, g, be, w_next)

  return pl.pallas_call(body, out_shape=out_shape)(*args)


def kernel(x, edge_index, W0, b0, g0, be0, W1, b1, g1, be1, W2, b2, g2, be2):
  src = edge_index[0]
  dst = edge_index[1]
  pad = E_PAD - E
  src_p = jnp.concatenate([src, jnp.zeros((pad,), jnp.int32)])
  pad_dst = N + jnp.arange(pad, dtype=jnp.int32) % (ACC_ROWS - N)
  dst_p = jnp.concatenate([dst, pad_dst])
  src2 = jnp.stack([src_p, src_p + N]).reshape(NC, NS, NBLK, BLK, CHUNK)
  dst_agg = dst_p.reshape(NS, NBLK, BLK, CHUNK)

  dst_deg = dst_p.reshape(NC, NS, DEG_NCHUNKS, DEG_CHUNK)

  zeroH = jnp.zeros((ZR, HALF), jnp.float32)

  deg2 = _sc_degree(dst_deg, zeroH)
  h0 = _tc_matmul0(x, W0)
  h2, dinv_row = _tc_scale_split(h0, deg2)

  params = [(b0, g0, be0, W1), (b1, g1, be1, W2), (b2, g2, be2, None)]
  for b, g, be, w_next in params:
    aggflat = _sc_aggregate(h2.reshape(NC * N, HALF), src2, dst_agg, zeroH)
    h2 = _tc_layer(aggflat.reshape(NC, ACC_ROWS, HALF), h2, dinv_row, b, g, be,
                   w_next)
  return h2


# speedup vs baseline: 1.1032x; 1.0405x over previous
import functools

import jax
import jax.numpy as jnp
from jax import lax
from jax.experimental import pallas as pl
from jax.experimental.pallas import tpu as pltpu
from jax.experimental.pallas import tpu_sc as plsc

N = 10000
E = 320000
D_IN = 128
D_H = 256
HALF = 128
EPS = 1e-5

NC = 2
NS = 16
CHUNK = 64
BLK = 32
NBLK = 10
N_CHUNKS = NBLK * BLK
EPT = N_CHUNKS * CHUNK
E_PAD = NS * EPT
ACC_ROWS = 10240
ZR = ACC_ROWS // NS

DEG_CHUNK = 128
DEG_EPT = E_PAD // (NC * NS)
DEG_NCHUNKS = DEG_EPT // DEG_CHUNK


def _sc_mesh():
  return plsc.VectorSubcoreMesh(
      core_axis_name="c", subcore_axis_name="s", num_cores=NC, num_subcores=NS)


@functools.partial(
    pl.kernel,
    out_type=jax.ShapeDtypeStruct((NC, ACC_ROWS, HALF), jnp.float32),
    mesh=_sc_mesh(),
    scratch_types=[
        pltpu.VMEM((DEG_NCHUNKS, DEG_CHUNK), jnp.int32),
        pltpu.VMEM((DEG_CHUNK, HALF), jnp.float32),
        pltpu.VMEM_SHARED((ACC_ROWS, HALF), jnp.float32),
        pltpu.SemaphoreType.DMA,
    ],
)
def _sc_degree(dst_hbm, zero_hbm, out_hbm, didx, ones, acc, ssem):
  c = lax.axis_index("c")
  s = lax.axis_index("s")

  pltpu.sync_copy(zero_hbm.at[pl.ds(0, ZR)], acc.at[pl.ds(s * ZR, ZR)])
  pltpu.sync_copy(dst_hbm.at[c, s], didx)

  @pl.loop(0, DEG_CHUNK)
  def _fill(r):
    for j in range(HALF // 16):
      ones[r, pl.ds(j * 16, 16)] = jnp.full((16,), 1.0, dtype=jnp.float32)

  plsc.subcore_barrier()

  for g in range(DEG_NCHUNKS):
    pltpu.async_copy(ones, acc.at[didx.at[g]], ssem, add=True)
  for g in range(DEG_NCHUNKS):
    pltpu.make_async_copy(ones, acc.at[didx.at[g]], ssem).wait()

  plsc.subcore_barrier()
  pltpu.sync_copy(acc.at[pl.ds(s * ZR, ZR)], out_hbm.at[c, pl.ds(s * ZR, ZR)])


@functools.partial(
    pl.kernel,
    out_type=jax.ShapeDtypeStruct((NC * ACC_ROWS, HALF), jnp.float32),
    mesh=_sc_mesh(),
    scratch_types=[
        pltpu.VMEM((2, BLK, CHUNK), jnp.int32),
        pltpu.VMEM((2, BLK, CHUNK), jnp.int32),
        pltpu.VMEM((4, CHUNK, HALF), jnp.float32),
        pltpu.VMEM_SHARED((ACC_ROWS, HALF), jnp.float32),
        pltpu.SemaphoreType.DMA,
        pltpu.SemaphoreType.DMA,
        pltpu.SemaphoreType.DMA,
        pltpu.SemaphoreType.DMA,
        pltpu.SemaphoreType.DMA,
        pltpu.SemaphoreType.DMA,
        pltpu.SemaphoreType.DMA,
        pltpu.SemaphoreType.DMA,
        pltpu.SemaphoreType.DMA,
        pltpu.SemaphoreType.DMA,
    ],
)
def _sc_aggregate(h2_hbm, src_hbm, dst_hbm, zero_hbm, out_hbm,
                  sidx, didx, rows, acc, gsem0, gsem1, gsem2, gsem3,
                  ssem0, ssem1, ssem2, ssem3, isem0, isem1):
  c = lax.axis_index("c")
  s = lax.axis_index("s")
  gsems = (gsem0, gsem1, gsem2, gsem3)
  ssems = (ssem0, ssem1, ssem2, ssem3)
  isems = (isem0, isem1)

  pltpu.sync_copy(zero_hbm.at[pl.ds(0, ZR)], acc.at[pl.ds(s * ZR, ZR)])
  pltpu.sync_copy(src_hbm.at[c, s, 0], sidx.at[0])
  pltpu.sync_copy(dst_hbm.at[s, 0], didx.at[0])
  pltpu.async_copy(src_hbm.at[c, s, 1], sidx.at[1], isem1)
  pltpu.async_copy(dst_hbm.at[s, 1], didx.at[1], isem1)
  plsc.subcore_barrier()
  for p in range(3):
    pltpu.async_copy(h2_hbm.at[sidx.at[0, p]], rows.at[p], gsems[p])

  @pl.loop(0, NBLK, step=2)
  def _outer(k0):
    for kb in range(2):
      blk = k0 + kb
      for j in range(BLK):
        b = j % 4
        pltpu.make_async_copy(h2_hbm.at[sidx.at[kb, j]], rows.at[b],
                              gsems[b]).wait()
        pltpu.async_copy(rows.at[b], acc.at[didx.at[kb, j]], ssems[b],
                         add=True)

        pb = (b + 3) % 4
        if j == 0:
          @pl.when(blk > 0)
          def _wait_prev_tail():
            pltpu.make_async_copy(rows.at[pb], acc.at[didx.at[1 - kb, 0]],
                                  ssems[pb]).wait()

          @pl.when((blk >= 1) & (blk + 1 < NBLK))
          def _refill():
            pltpu.async_copy(src_hbm.at[c, s, blk + 1], sidx.at[1 - kb],
                             isems[1 - kb])
            pltpu.async_copy(dst_hbm.at[s, blk + 1], didx.at[1 - kb],
                             isems[1 - kb])
        else:
          pltpu.make_async_copy(rows.at[pb], acc.at[didx.at[kb, j - 1]],
                                ssems[pb]).wait()

        if j < BLK - 3:
          pltpu.async_copy(h2_hbm.at[sidx.at[kb, j + 3]], rows.at[pb],
                           gsems[pb])
        else:
          nj = j + 3 - BLK

          @pl.when(blk + 1 < NBLK)
          def _prefetch_next_block():
            if nj == 0:
              pltpu.make_async_copy(src_hbm.at[c, s, blk + 1],
                                    sidx.at[1 - kb], isems[1 - kb]).wait()
              pltpu.make_async_copy(dst_hbm.at[s, blk + 1],
                                    didx.at[1 - kb], isems[1 - kb]).wait()
            pltpu.async_copy(h2_hbm.at[sidx.at[1 - kb, nj]], rows.at[pb],
                             gsems[pb])

  pltpu.make_async_copy(rows.at[(BLK - 1) % 4], acc.at[didx.at[1, BLK - 1]],
                        ssems[(BLK - 1) % 4]).wait()
  plsc.subcore_barrier()

  pltpu.sync_copy(acc.at[pl.ds(s * ZR, ZR)],
                  out_hbm.at[pl.ds(c * ACC_ROWS + s * ZR, ZR)])


def _tc_matmul0(x, w0):
  def body(x_ref, w_ref, h_ref):
    h_ref[...] = jnp.dot(x_ref[...], w_ref[...],
                         preferred_element_type=jnp.float32)

  return pl.pallas_call(
      body,
      out_shape=jax.ShapeDtypeStruct((N, D_H), jnp.float32),
  )(x, w0)


def _tc_scale_split(h0, deg2):
  def body(h_ref, deg_ref, h2_ref, dinv_ref):
    deg = deg_ref[0, :N, 0] + deg_ref[1, :N, 0] + 1.0
    dinv = lax.rsqrt(deg)
    hp = h_ref[...] * dinv[:, None]
    h2_ref[0, :, :] = hp[:, :HALF]
    h2_ref[1, :, :] = hp[:, HALF:]
    dinv_ref[...] = jnp.broadcast_to(dinv[None, :], (8, N))

  return pl.pallas_call(
      body,
      out_shape=(jax.ShapeDtypeStruct((NC, N, HALF), jnp.float32),
                 jax.ShapeDtypeStruct((8, N), jnp.float32)),
  )(h0, deg2)


def _tc_layer(agg2, h2prev, dinv_row, b, g, be, w_next):
  final = w_next is None

  def body(agg_ref, hprev_ref, dinv_ref, b_ref, g_ref, be_ref, *rest):
    if final:
      (out_ref,) = rest
    else:
      w_ref, out_ref = rest
    dinv = dinv_ref[0, :]

    halves = []
    for h_i in range(NC):
      z = (agg_ref[h_i, :N, :] + hprev_ref[h_i]) * dinv[:, None]
      z = z + b_ref[pl.ds(h_i * HALF, HALF)][None, :]
      mu = jnp.mean(z, axis=0)
      zc = z - mu[None, :]
      var = jnp.mean(zc * zc, axis=0)
      zn = (g_ref[pl.ds(h_i * HALF, HALF)][None, :] * zc
            * lax.rsqrt(var + EPS)[None, :]
            + be_ref[pl.ds(h_i * HALF, HALF)][None, :])
      halves.append(jnp.maximum(zn, 0.0))

    if final:
      out_ref[:, :HALF] = halves[0]
      out_ref[:, HALF:] = halves[1]
    else:
      hn = (jnp.dot(halves[0], w_ref[:HALF, :],
                    preferred_element_type=jnp.float32)
            + jnp.dot(halves[1], w_ref[HALF:, :],
                      preferred_element_type=jnp.float32))
      hp = hn * dinv[:, None]
      out_ref[0, :, :] = hp[:, :HALF]
      out_ref[1, :, :] = hp[:, HALF:]

  if final:
    out_shape = jax.ShapeDtypeStruct((N, D_H), jnp.float32)
    args = (agg2, h2prev, dinv_row, b, g, be)
  else:
    out_shape = jax.ShapeDtypeStruct((NC, N, HALF), jnp.float32)
    args = (agg2, h2prev, dinv_row, b, g, be, w_next)

  return pl.pallas_call(body, out_shape=out_shape)(*args)


def kernel(x, edge_index, W0, b0, g0, be0, W1, b1, g1, be1, W2, b2, g2, be2):
  src = edge_index[0]
  dst = edge_index[1]
  pad = E_PAD - E
  src_p = jnp.concatenate([src, jnp.zeros((pad,), jnp.int32)])
  pad_dst = N + jnp.arange(pad, dtype=jnp.int32) % (ACC_ROWS - N)
  dst_p = jnp.concatenate([dst, pad_dst])
  src2 = jnp.stack([src_p, src_p + N]).reshape(NC, NS, NBLK, BLK, CHUNK)
  dst_agg = dst_p.reshape(NS, NBLK, BLK, CHUNK)

  dst_deg = dst_p.reshape(NC, NS, DEG_NCHUNKS, DEG_CHUNK)

  zeroH = jnp.zeros((ZR, HALF), jnp.float32)

  deg2 = _sc_degree(dst_deg, zeroH)
  h0 = _tc_matmul0(x, W0)
  h2, dinv_row = _tc_scale_split(h0, deg2)

  params = [(b0, g0, be0, W1), (b1, g1, be1, W2), (b2, g2, be2, None)]
  for b, g, be, w_next in params:
    aggflat = _sc_aggregate(h2.reshape(NC * N, HALF), src2, dst_agg, zeroH)
    h2 = _tc_layer(aggflat.reshape(NC, ACC_ROWS, HALF), h2, dinv_row, b, g, be,
                   w_next)
  return h2
